# baseline (speedup 1.0000x reference)
import jax
import jax.numpy as jnp
from jax import lax
from jax.experimental import pallas as pl
from jax.experimental.pallas import tpu as pltpu
from jax.experimental.pallas import tpu_sc as plsc

_D = 64
_NUM_ROWS = 1000000
_B_TOTAL = 16384 * 50

_info = plsc.get_sparse_core_info()
_NC = _info.num_cores
_NS = _info.num_subcores
_NW = _NC * _NS
_B_PER_W = _B_TOTAL // _NW
_CHUNK = 800
_NCHUNK = _B_PER_W // _CHUNK
_NPAIR = _NCHUNK // 2


def _gather_body(idx_hbm, table_hbm, out_hbm, idx_v, rows0, rows1, sg0, sg1):
    wid = lax.axis_index("s") * _NC + lax.axis_index("c")
    base = wid * _B_PER_W
    rows = (rows0, rows1)
    sems = (sg0, sg1)

    pltpu.sync_copy(idx_hbm.at[pl.ds(base, _B_PER_W)], idx_v)

    def gather(c, b):
        return pltpu.make_async_copy(
            table_hbm.at[idx_v.at[pl.ds(c * _CHUNK, _CHUNK)]], rows[b], sems[b]
        )

    gather(0, 0).start()
    gather(1, 1).start()

    def body(p, carry):
        for b in range(2):
            c = 2 * p + b
            gather(c, b).wait()
            pltpu.sync_copy(rows[b], out_hbm.at[pl.ds(base + c * _CHUNK, _CHUNK)])

            @pl.when(p + 1 < _NPAIR)
            def _():
                gather(c + 2, b).start()

        return carry

    lax.fori_loop(0, _NPAIR, body, 0)


_BATCH = 16384
_HIST = 50
_BPW = _BATCH // _NW
_NBB = _BPW // 128
_NQ = (_HIST * _D) // 128


def _out_transpose_body(src_hbm, out_hbm, in_v, out_v):
    wid = lax.axis_index("s") * _NC + lax.axis_index("c")
    b_base = wid * _BPW

    def chunk(it, carry):
        bb = it // _NQ
        q = it % _NQ
        b0 = b_base + bb * 128
        pltpu.sync_copy(src_hbm.at[pl.ds(b0, 128), pl.ds(q * 128, 128)], in_v)

        def xp(x, c2):
            for g in range(8):
                rows = lax.iota(jnp.int32, 16) + g * 16
                cols = jnp.full((16,), x, jnp.int32)
                vals = plsc.load_gather(in_v, [rows, cols])
                out_v[x, pl.ds(g * 16, 16)] = vals
            return c2

        lax.fori_loop(0, 128, xp, 0)
        pltpu.sync_copy(out_v.at[pl.ds(0, 64)], out_hbm.at[2 * q, :, pl.ds(b0, 128)])
        pltpu.sync_copy(out_v.at[pl.ds(64, 64)], out_hbm.at[2 * q + 1, :, pl.ds(b0, 128)])
        return carry

    lax.fori_loop(0, _NBB * _NQ, chunk, 0)


@jax.jit
def kernel(indices, embeddings):
    b, h = indices.shape
    idx_flat = indices.reshape(-1).astype(jnp.int32)
    table = jax.lax.optimization_barrier(
        embeddings.reshape(_NUM_ROWS // 2, 2 * _D)
    ).reshape(_NUM_ROWS, _D)
    mesh = plsc.VectorSubcoreMesh(core_axis_name="c", subcore_axis_name="s")
    out = pl.kernel(
        _gather_body,
        mesh=mesh,
        out_type=jax.ShapeDtypeStruct((_B_TOTAL, _D), jnp.float32),
        scratch_types=[
            pltpu.VMEM((_B_PER_W,), jnp.int32),
            pltpu.VMEM((_CHUNK, _D), jnp.float32),
            pltpu.VMEM((_CHUNK, _D), jnp.float32),
            pltpu.SemaphoreType.DMA,
            pltpu.SemaphoreType.DMA,
        ],
        compiler_params=pltpu.CompilerParams(use_tc_tiling_on_sc=False),
    )(idx_flat, table)
    out3 = pl.kernel(
        _out_transpose_body,
        mesh=mesh,
        out_type=jax.ShapeDtypeStruct((_HIST, _D, _BATCH), jnp.float32),
        scratch_types=[
            pltpu.VMEM((128, 128), jnp.float32),
            pltpu.VMEM((128, 128), jnp.float32),
        ],
        compiler_params=pltpu.CompilerParams(
            use_tc_tiling_on_sc=True, needs_layout_passes=False
        ),
    )(out.reshape(_BATCH, _HIST * _D))
    return jnp.transpose(out3, (2, 0, 1))


# speedup vs baseline: 1.2981x; 1.2981x over previous
import jax
import jax.numpy as jnp
from jax import lax
from jax.experimental import pallas as pl
from jax.experimental.pallas import tpu as pltpu
from jax.experimental.pallas import tpu_sc as plsc

_D = 64
_NUM_ROWS = 1000000
_BATCH = 16384
_HIST = 50

_info = plsc.get_sparse_core_info()
_NC = _info.num_cores
_NS = _info.num_subcores
_NW = _NC * _NS
_BPW = _BATCH // _NW
_NBB = _BPW // 128
_NJ = _HIST * _NBB


def _body(idx_hbm, table_hbm, out_hbm, idx_v, idxt_v, rows0, rows1, t0, t1,
          sg0, sg1, sw0, sw1):
    wid = lax.axis_index("s") * _NC + lax.axis_index("c")
    rows = (rows0, rows1)
    tv = (t0, t1)
    sg = (sg0, sg1)
    sw = (sw0, sw1)
    iota = lax.iota(jnp.int32, 16)
    iota_h = iota * _HIST

    pltpu.sync_copy(idx_hbm.at[pl.ds(wid * _BPW * _HIST, _BPW * _HIST)], idx_v)

    def build(j, carry):
        h = j // _NBB
        bb = j % _NBB
        for g in range(8):
            src = iota_h + ((bb * 128 + g * 16) * _HIST + h)
            idxt_v[j, pl.ds(g * 16, 16)] = plsc.load_gather(idx_v, [src])
        return carry

    lax.fori_loop(0, _NJ, build, 0)

    def gather(j, p):
        return pltpu.make_async_copy(
            table_hbm.at[idxt_v.at[j]], rows[p], sg[p]
        )

    def write(j, p):
        h = j // _NBB
        btg = wid * _NBB + j % _NBB
        return pltpu.make_async_copy(
            tv[p], out_hbm.at[j // _NBB, :, wid * _NBB + j % _NBB, :, :], sw[p]
        )

    gather(0, 0).start()
    gather(1, 1).start()

    def step(jp, carry):
        for p in range(2):
            j = 2 * jp + p

            @pl.when(j >= 2)
            def _():
                write(j - 2, p).wait()

            gather(j, p).wait()

            def xp(d, c2):
                dt = d // 8
                ds = d % 8
                for g in range(8):
                    vals = plsc.load_gather(
                        rows[p], [iota + g * 16, jnp.full((16,), d, jnp.int32)]
                    )
                    tv[p][dt, ds, pl.ds(g * 16, 16)] = vals
                return c2

            lax.fori_loop(0, _D, xp, 0)
            write(j, p).start()

            @pl.when(j + 2 < _NJ)
            def _():
                gather(j + 2, p).start()

        return carry

    lax.fori_loop(0, _NJ // 2, step, 0)
    write(_NJ - 2, 0).wait()
    write(_NJ - 1, 1).wait()


@jax.jit
def kernel(indices, embeddings):
    b, h = indices.shape
    idx_flat = indices.reshape(-1).astype(jnp.int32)
    table = jax.lax.optimization_barrier(
        embeddings.reshape(_NUM_ROWS // 2, 2 * _D)
    ).reshape(_NUM_ROWS, _D)
    mesh = plsc.VectorSubcoreMesh(core_axis_name="c", subcore_axis_name="s")
    out5 = pl.kernel(
        _body,
        mesh=mesh,
        out_type=jax.ShapeDtypeStruct((_HIST, 8, _BATCH // 128, 8, 128),
                                      jnp.float32),
        scratch_types=[
            pltpu.VMEM((_BPW * _HIST,), jnp.int32),
            pltpu.VMEM((_NJ, 128), jnp.int32),
            pltpu.VMEM((128, _D), jnp.float32),
            pltpu.VMEM((128, _D), jnp.float32),
            pltpu.VMEM((8, 8, 128), jnp.float32),
            pltpu.VMEM((8, 8, 128), jnp.float32),
            pltpu.SemaphoreType.DMA,
            pltpu.SemaphoreType.DMA,
            pltpu.SemaphoreType.DMA,
            pltpu.SemaphoreType.DMA,
        ],
        compiler_params=pltpu.CompilerParams(
            use_tc_tiling_on_sc=False, needs_layout_passes=False
        ),
    )(idx_flat, table)
    x = jnp.transpose(out5, (0, 1, 3, 2, 4)).reshape(_HIST, _D, _BATCH)
    return jnp.transpose(x, (2, 0, 1))


# speedup vs baseline: 1.4866x; 1.1452x over previous
import jax
import jax.numpy as jnp
from jax import lax
from jax.experimental import pallas as pl
from jax.experimental.pallas import tpu as pltpu
from jax.experimental.pallas import tpu_sc as plsc

_D = 64
_NUM_ROWS = 1000000
_BATCH = 16384
_HIST = 50

_info = plsc.get_sparse_core_info()
_NC = _info.num_cores
_NS = _info.num_subcores
_NW = _NC * _NS
_BPW = _BATCH // _NW
_NBB = _BPW // 128
_NJ = _HIST * _NBB


def _body(idx_hbm, table_hbm, out_hbm, idx_v, idxt_v, rows0, rows1, t0, t1,
          sg0, sg1, sw0, sw1):
    wid = lax.axis_index("s") * _NC + lax.axis_index("c")
    rows = (rows0, rows1)
    tv = (t0, t1)
    sg = (sg0, sg1)
    sw = (sw0, sw1)
    iota = lax.iota(jnp.int32, 16)
    iota_h = iota * _HIST
    dtv = [(iota + 16 * q) // 8 for q in range(4)]
    dsv = [(iota + 16 * q) % 8 for q in range(4)]

    pltpu.sync_copy(idx_hbm.at[pl.ds(wid * _BPW * _HIST, _BPW * _HIST)], idx_v)

    def build(j, carry):
        h = j // _NBB
        bb = j % _NBB
        for g in range(8):
            src = iota_h + ((bb * 128 + g * 16) * _HIST + h)
            idxt_v[j, pl.ds(g * 16, 16)] = plsc.load_gather(idx_v, [src])
        return carry

    lax.fori_loop(0, _NJ, build, 0)

    def gather(j, p):
        return pltpu.make_async_copy(
            table_hbm.at[idxt_v.at[j]], rows[p], sg[p]
        )

    def write(j, p):
        h = j // _NBB
        btg = wid * _NBB + j % _NBB
        return pltpu.make_async_copy(
            tv[p], out_hbm.at[j // _NBB, :, wid * _NBB + j % _NBB, :, :], sw[p]
        )

    gather(0, 0).start()
    gather(1, 1).start()

    def step(jp, carry):
        for p in range(2):
            j = 2 * jp + p

            @pl.when(j >= 2)
            def _():
                write(j - 2, p).wait()

            gather(j, p).wait()

            def xp(i, c2):
                for o in range(2):
                    bl = 2 * i + o
                    blv = jnp.full((16,), bl, jnp.int32)
                    for q in range(4):
                        vals = rows[p][bl, pl.ds(16 * q, 16)]
                        plsc.store_scatter(tv[p], [dtv[q], dsv[q], blv], vals)
                return c2

            lax.fori_loop(0, 64, xp, 0)
            write(j, p).start()

            @pl.when(j + 2 < _NJ)
            def _():
                gather(j + 2, p).start()

        return carry

    lax.fori_loop(0, _NJ // 2, step, 0)
    write(_NJ - 2, 0).wait()
    write(_NJ - 1, 1).wait()


@jax.jit
def kernel(indices, embeddings):
    b, h = indices.shape
    idx_flat = indices.reshape(-1).astype(jnp.int32)
    table = jax.lax.optimization_barrier(
        embeddings.reshape(_NUM_ROWS // 2, 2 * _D)
    ).reshape(_NUM_ROWS, _D)
    mesh = plsc.VectorSubcoreMesh(core_axis_name="c", subcore_axis_name="s")
    out5 = pl.kernel(
        _body,
        mesh=mesh,
        out_type=jax.ShapeDtypeStruct((_HIST, 8, _BATCH // 128, 8, 128),
                                      jnp.float32),
        scratch_types=[
            pltpu.VMEM((_BPW * _HIST,), jnp.int32),
            pltpu.VMEM((_NJ, 128), jnp.int32),
            pltpu.VMEM((128, _D), jnp.float32),
            pltpu.VMEM((128, _D), jnp.float32),
            pltpu.VMEM((8, 8, 128), jnp.float32),
            pltpu.VMEM((8, 8, 128), jnp.float32),
            pltpu.SemaphoreType.DMA,
            pltpu.SemaphoreType.DMA,
            pltpu.SemaphoreType.DMA,
            pltpu.SemaphoreType.DMA,
        ],
        compiler_params=pltpu.CompilerParams(
            use_tc_tiling_on_sc=False, needs_layout_passes=False
        ),
    )(idx_flat, table)
    x = jnp.transpose(out5, (0, 1, 3, 2, 4)).reshape(_HIST, _D, _BATCH)
    return jnp.transpose(x, (2, 0, 1))


# speedup vs baseline: 1.5322x; 1.0307x over previous
import jax
import jax.numpy as jnp
from jax import lax
from jax.experimental import pallas as pl
from jax.experimental.pallas import tpu as pltpu
from jax.experimental.pallas import tpu_sc as plsc

_D = 64
_NUM_ROWS = 1000000
_BATCH = 16384
_HIST = 50

_info = plsc.get_sparse_core_info()
_NC = _info.num_cores
_NS = _info.num_subcores
_NW = _NC * _NS
_BPW = _BATCH // _NW
_NBB = _BPW // 128
_NJ = _HIST * _NBB


def _body(idx_hbm, table_hbm, out_hbm, idx_v, idxt_v, rows0, rows1, t0, t1,
          sg0, sg1, sw0, sw1):
    wid = lax.axis_index("s") * _NC + lax.axis_index("c")
    rows = (rows0, rows1)
    tv = (t0, t1)
    sg = (sg0, sg1)
    sw = (sw0, sw1)
    iota = lax.iota(jnp.int32, 16)
    iota_h = iota * _HIST
    dtv = [(iota + 16 * q) // 8 for q in range(4)]
    dsv = [(iota + 16 * q) % 8 for q in range(4)]

    pltpu.sync_copy(idx_hbm.at[pl.ds(wid * _BPW * _HIST, _BPW * _HIST)], idx_v)

    def build(j, carry):
        h = j // _NBB
        bb = j % _NBB
        for g in range(8):
            src = iota_h + ((bb * 128 + g * 16) * _HIST + h)
            idxt_v[j, pl.ds(g * 16, 16)] = plsc.load_gather(idx_v, [src])
        return carry

    lax.fori_loop(0, _NJ, build, 0)

    def gather(j, p):
        return pltpu.make_async_copy(
            table_hbm.at[idxt_v.at[j]], rows[p], sg[p]
        )

    def write(j, p):
        h = j // _NBB
        btg = wid * _NBB + j % _NBB
        return pltpu.make_async_copy(
            tv[p], out_hbm.at[j // _NBB, :, wid * _NBB + j % _NBB, :, :], sw[p]
        )

    gather(0, 0).start()
    gather(1, 1).start()

    def step(jp, carry):
        for p in range(2):
            j = 2 * jp + p

            @pl.when(j >= 2)
            def _():
                write(j - 2, p).wait()

            gather(j, p).wait()

            def xp(i, blv):
                blvs = [blv + o for o in range(4)]
                vals = [
                    rows[p][4 * i + o, pl.ds(16 * q, 16)]
                    for o in range(4)
                    for q in range(4)
                ]
                for o in range(4):
                    for q in range(4):
                        plsc.store_scatter(
                            tv[p], [dtv[q], dsv[q], blvs[o]], vals[4 * o + q]
                        )
                return blv + 4

            lax.fori_loop(0, 32, xp, jnp.zeros((16,), jnp.int32))
            write(j, p).start()

            @pl.when(j + 2 < _NJ)
            def _():
                gather(j + 2, p).start()

        return carry

    lax.fori_loop(0, _NJ // 2, step, 0)
    write(_NJ - 2, 0).wait()
    write(_NJ - 1, 1).wait()


@jax.jit
def kernel(indices, embeddings):
    b, h = indices.shape
    idx_flat = indices.reshape(-1).astype(jnp.int32)
    table = jax.lax.optimization_barrier(
        embeddings.reshape(_NUM_ROWS // 2, 2 * _D)
    ).reshape(_NUM_ROWS, _D)
    mesh = plsc.VectorSubcoreMesh(core_axis_name="c", subcore_axis_name="s")
    out5 = pl.kernel(
        _body,
        mesh=mesh,
        out_type=jax.ShapeDtypeStruct((_HIST, 8, _BATCH // 128, 8, 128),
                                      jnp.float32),
        scratch_types=[
            pltpu.VMEM((_BPW * _HIST,), jnp.int32),
            pltpu.VMEM((_NJ, 128), jnp.int32),
            pltpu.VMEM((128, _D), jnp.float32),
            pltpu.VMEM((128, _D), jnp.float32),
            pltpu.VMEM((8, 8, 128), jnp.float32),
            pltpu.VMEM((8, 8, 128), jnp.float32),
            pltpu.SemaphoreType.DMA,
            pltpu.SemaphoreType.DMA,
            pltpu.SemaphoreType.DMA,
            pltpu.SemaphoreType.DMA,
        ],
        compiler_params=pltpu.CompilerParams(
            use_tc_tiling_on_sc=False, needs_layout_passes=False
        ),
    )(idx_flat, table)
    x = jnp.transpose(out5, (0, 1, 3, 2, 4)).reshape(_HIST, _D, _BATCH)
    return jnp.transpose(x, (2, 0, 1))


# speedup vs baseline: 2.7047x; 1.7653x over previous
import jax
import jax.numpy as jnp
from jax import lax
from jax.experimental import pallas as pl
from jax.experimental.pallas import tpu as pltpu
from jax.experimental.pallas import tpu_sc as plsc

_D = 64
_NUM_ROWS = 1000000
_BATCH = 16384
_HIST = 50

_info = plsc.get_sparse_core_info()
_NC = _info.num_cores
_NS = _info.num_subcores
_NW = _NC * _NS
_BPW = _BATCH // _NW
_NBB = _BPW // 128
_NJ = _HIST * _NBB


def _body(idx_hbm, table_hbm, out_hbm, idx_v, idxt_v, rows0, rows1, t0, t1,
          sg0, sg1, sw0, sw1):
    wid = lax.axis_index("s") * _NC + lax.axis_index("c")
    rows = (rows0, rows1)
    tv = (t0, t1)
    sg = (sg0, sg1)
    sw = (sw0, sw1)
    iota = lax.iota(jnp.int32, 16)
    iota_h = iota * _HIST
    dtv = [(iota + 16 * q) // 8 for q in range(4)]
    dsv = [(iota + 16 * q) % 8 for q in range(4)]

    pltpu.sync_copy(idx_hbm.at[pl.ds(wid * _BPW * _HIST, _BPW * _HIST)], idx_v)

    def build(j, carry):
        h = j // _NBB
        bb = j % _NBB
        for g in range(8):
            src = iota_h + ((bb * 128 + g * 16) * _HIST + h)
            idxt_v[j, pl.ds(g * 16, 16)] = plsc.load_gather(idx_v, [src])
        return carry

    lax.fori_loop(0, _NJ, build, 0)

    def gather(j, p):
        return pltpu.make_async_copy(
            table_hbm.at[idxt_v.at[j]], rows[p], sg[p]
        )

    def write(j, p):
        h = j // _NBB
        btg = wid * _NBB + j % _NBB
        return pltpu.make_async_copy(
            tv[p].at[:, :, pl.ds(0, 128)],
            out_hbm.at[j // _NBB, :, wid * _NBB + j % _NBB, :, :], sw[p]
        )

    gather(0, 0).start()
    gather(1, 1).start()

    def step(jp, carry):
        for p in range(2):
            j = 2 * jp + p

            @pl.when(j >= 2)
            def _():
                write(j - 2, p).wait()

            gather(j, p).wait()

            def xp(i, blv):
                blvs = [blv + o for o in range(4)]
                vals = [
                    rows[p][4 * i + o, pl.ds(16 * q, 16)]
                    for o in range(4)
                    for q in range(4)
                ]
                for o in range(4):
                    for q in range(4):
                        plsc.store_scatter(
                            tv[p], [dtv[q], dsv[q], blvs[o]], vals[4 * o + q]
                        )
                return blv + 4

            lax.fori_loop(0, 32, xp, jnp.zeros((16,), jnp.int32))
            write(j, p).start()

            @pl.when(j + 2 < _NJ)
            def _():
                gather(j + 2, p).start()

        return carry

    lax.fori_loop(0, _NJ // 2, step, 0)
    write(_NJ - 2, 0).wait()
    write(_NJ - 1, 1).wait()


@jax.jit
def kernel(indices, embeddings):
    b, h = indices.shape
    idx_flat = indices.reshape(-1).astype(jnp.int32)
    table = jax.lax.optimization_barrier(
        embeddings.reshape(_NUM_ROWS // 2, 2 * _D)
    ).reshape(_NUM_ROWS, _D)
    mesh = plsc.VectorSubcoreMesh(core_axis_name="c", subcore_axis_name="s")
    out5 = pl.kernel(
        _body,
        mesh=mesh,
        out_type=jax.ShapeDtypeStruct((_HIST, 8, _BATCH // 128, 8, 128),
                                      jnp.float32),
        scratch_types=[
            pltpu.VMEM((_BPW * _HIST,), jnp.int32),
            pltpu.VMEM((_NJ, 128), jnp.int32),
            pltpu.VMEM((128, _D), jnp.float32),
            pltpu.VMEM((128, _D), jnp.float32),
            pltpu.VMEM((8, 8, 129), jnp.float32),
            pltpu.VMEM((8, 8, 129), jnp.float32),
            pltpu.SemaphoreType.DMA,
            pltpu.SemaphoreType.DMA,
            pltpu.SemaphoreType.DMA,
            pltpu.SemaphoreType.DMA,
        ],
        compiler_params=pltpu.CompilerParams(
            use_tc_tiling_on_sc=False, needs_layout_passes=False
        ),
    )(idx_flat, table)
    x = jnp.transpose(out5, (0, 1, 3, 2, 4)).reshape(_HIST, _D, _BATCH)
    return jnp.transpose(x, (2, 0, 1))
